# Initial kernel scaffold; baseline (speedup 1.0000x reference)
import jax
import jax.numpy as jnp
from jax.experimental import pallas as pl


def kernel(emb, gcn1_W, gcn1_b, node1_W, node1_b, edge1_w, edge1_b, edge2_w, edge2_b, gcn2_W, gcn2_b, edge_index):
    raise NotImplementedError("write your pallas kernel here")


# speedup vs baseline: 1.2658x; 1.2658x over previous
import functools

import jax
import jax.numpy as jnp
from jax import lax
from jax.experimental import pallas as pl


def _mm_body(x_ref, w_ref, o_ref):
    o_ref[...] = jnp.dot(x_ref[...], w_ref[...], preferred_element_type=jnp.float32)


def _mm(x, w, bn=1000):
    n, k = x.shape
    m = w.shape[1]
    grid = n // bn
    return pl.pallas_call(
        _mm_body,
        grid=(grid,),
        in_specs=[
            pl.BlockSpec((bn, k), lambda i: (i, 0)),
            pl.BlockSpec((k, m), lambda i: (0, 0)),
        ],
        out_specs=pl.BlockSpec((bn, m), lambda i: (i, 0)),
        out_shape=jax.ShapeDtypeStruct((n, m), jnp.float32),
    )(x, w)


def kernel(emb, gcn1_W, gcn1_b, node1_W, node1_b, edge1_w, edge1_b, edge2_w,
           edge2_b, gcn2_W, gcn2_b, edge_index):
    n = emb.shape[0]
    h_dim = gcn1_W.shape[1]
    src = edge_index[0]
    dst = edge_index[1]
    e = src.shape[0]
    f32 = jnp.float32

    cnt = jax.ops.segment_sum(jnp.ones((e,), f32), dst, num_segments=n)
    deg = cnt + 1.0
    dinv = lax.rsqrt(deg)

    xw1 = _mm(emb, gcn1_W)
    u = xw1 * dinv[:, None]
    s1 = jax.ops.segment_sum(u[src], dst, num_segments=n)
    pre = dinv[:, None] * s1 + (dinv * dinv)[:, None] * xw1 + gcn1_b
    nrm = jnp.sqrt(jnp.sum(pre * pre, axis=-1, keepdims=True))
    h = pre / jnp.maximum(nrm, 1e-12)

    p1 = h @ edge1_w[:h_dim]
    q1 = h @ edge1_w[h_dim:]
    ev1 = jax.nn.sigmoid(p1[src] + q1[dst] + edge1_b[0])
    a2 = jax.ops.segment_sum(h[src] * ev1[:, None], dst, num_segments=n)
    na = jax.nn.relu(_mm(a2, node1_W) + node1_b)

    p2 = na @ edge2_w[:h_dim]
    q2 = na @ edge2_w[h_dim:]
    ev2 = jax.nn.sigmoid(p2[src] + q2[dst] + edge2_b[0])
    a3 = jax.ops.segment_sum(na[src] * ev2[:, None], dst, num_segments=n)
    h2 = jax.nn.relu(_mm(a3, node1_W) + node1_b)

    deg2 = jax.ops.segment_sum(ev2, dst, num_segments=n) + 1.0
    dinv2 = lax.rsqrt(deg2)
    xw2 = _mm(h2, gcn2_W)
    u2 = xw2 * dinv2[:, None]
    s4 = jax.ops.segment_sum(u2[src] * ev2[:, None], dst, num_segments=n)
    out = dinv2[:, None] * s4 + (dinv2 * dinv2)[:, None] * xw2 + gcn2_b
    return out


# speedup vs baseline: 11.3619x; 8.9757x over previous
import functools

import jax
import jax.numpy as jnp
from jax import lax
from jax.experimental import pallas as pl
from jax.experimental.pallas import tpu as pltpu
from jax.experimental.pallas import tpu_sc as plsc

NC = 2
NS = 16
D = 128


def _fill1d(ref, n, val):
    v = jnp.full((16,), val, jnp.float32)

    def body(i, carry):
        ref[pl.ds(i * 16, 16)] = v
        return carry

    lax.fori_loop(0, n // 16, body, 0)


def _fill2d(ref, rows, cols, val):
    v = jnp.full((16,), val, jnp.float32)

    def body(i, carry):
        for cb in range(cols // 16):
            ref[i, pl.ds(cb * 16, 16)] = v
        return carry

    lax.fori_loop(0, rows, body, 0)


def _make_degcount(e, n_pad):
    epw = e // (NC * NS)
    per_tile = n_pad // NS
    mesh = plsc.VectorSubcoreMesh(core_axis_name="c", subcore_axis_name="s")

    @functools.partial(
        pl.kernel, mesh=mesh,
        out_type=jax.ShapeDtypeStruct((NC * n_pad,), jnp.float32),
        scratch_types=[
            pltpu.VMEM((epw,), jnp.int32),
            pltpu.VMEM((epw,), jnp.float32),
            pltpu.VMEM((per_tile,), jnp.float32),
            pltpu.VMEM_SHARED((n_pad,), jnp.float32),
        ],
    )
    def k(dst_hbm, out_hbm, dst_v, ones_v, zed_v, acc_sh):
        c = lax.axis_index("c")
        s = lax.axis_index("s")
        wid = c * NS + s
        _fill1d(zed_v, per_tile, 0.0)
        pltpu.sync_copy(zed_v, acc_sh.at[pl.ds(s * per_tile, per_tile)])
        _fill1d(ones_v, epw, 1.0)
        plsc.subcore_barrier()
        base = wid * epw
        pltpu.sync_copy(dst_hbm.at[pl.ds(base, epw)], dst_v)
        pltpu.sync_copy(ones_v, acc_sh.at[dst_v], add=True)
        plsc.subcore_barrier()
        pltpu.sync_copy(acc_sh.at[pl.ds(s * per_tile, per_tile)],
                        out_hbm.at[pl.ds(c * n_pad + s * per_tile, per_tile)])

    return k


def _make_gate(n, e, with_deg, n_pad):
    epw = e // (NC * NS)
    per_tile = n_pad // NS
    mesh = plsc.VectorSubcoreMesh(core_axis_name="c", subcore_axis_name="s")
    out_type = [jax.ShapeDtypeStruct((e,), jnp.float32)]
    del n
    scratch = [
        pltpu.VMEM((epw,), jnp.int32),
        pltpu.VMEM((epw,), jnp.int32),
        pltpu.VMEM((epw,), jnp.float32),
        pltpu.VMEM((epw,), jnp.float32),
        pltpu.VMEM((epw,), jnp.float32),
        pltpu.VMEM((16,), jnp.float32),
        pltpu.SemaphoreType.DMA,
    ]
    if with_deg:
        out_type.append(jax.ShapeDtypeStruct((NC * n_pad,), jnp.float32))
        scratch.append(pltpu.VMEM((per_tile,), jnp.float32))
        scratch.append(pltpu.VMEM_SHARED((n_pad,), jnp.float32))

    @functools.partial(pl.kernel, mesh=mesh, out_type=tuple(out_type),
                       scratch_types=scratch)
    def k(p_hbm, q_hbm, b_hbm, src_hbm, dst_hbm, *rest):
        if with_deg:
            (ev_hbm, deg_hbm, src_v, dst_v, pv_v, qv_v, ev_v, b_v, sem,
             zed_v, acc_sh) = rest
        else:
            ev_hbm, src_v, dst_v, pv_v, qv_v, ev_v, b_v, sem = rest
        c = lax.axis_index("c")
        s = lax.axis_index("s")
        wid = c * NS + s
        base = wid * epw
        if with_deg:
            _fill1d(zed_v, per_tile, 0.0)
            pltpu.sync_copy(zed_v, acc_sh.at[pl.ds(s * per_tile, per_tile)])
        pltpu.sync_copy(b_hbm, b_v)
        pltpu.sync_copy(src_hbm.at[pl.ds(base, epw)], src_v)
        pltpu.sync_copy(dst_hbm.at[pl.ds(base, epw)], dst_v)
        pltpu.async_copy(p_hbm.at[src_v], pv_v, sem).wait()
        pltpu.async_copy(q_hbm.at[dst_v], qv_v, sem).wait()
        bb = b_v[...]

        def body(g, carry):
            sl = pl.ds(g * 16, 16)
            x = pv_v[sl] + qv_v[sl] + bb
            ev_v[sl] = 1.0 / (1.0 + jnp.exp(-x))
            return carry

        lax.fori_loop(0, epw // 16, body, 0)
        pltpu.sync_copy(ev_v, ev_hbm.at[pl.ds(base, epw)])
        if with_deg:
            plsc.subcore_barrier()
            pltpu.sync_copy(ev_v, acc_sh.at[dst_v], add=True)
            plsc.subcore_barrier()
            pltpu.sync_copy(acc_sh.at[pl.ds(s * per_tile, per_tile)],
                            deg_hbm.at[pl.ds(c * n_pad + s * per_tile, per_tile)])

    return k


def _make_rowpass(n_pad, e, weighted, split_edges, chunk=320):
    ept = e // (NC * NS) if split_edges else e // NS
    n_iters = ept // chunk
    rpt = n_pad // NS
    mesh = plsc.VectorSubcoreMesh(core_axis_name="c", subcore_axis_name="s")

    @functools.partial(
        pl.kernel, mesh=mesh,
        out_type=jax.ShapeDtypeStruct((NC, n_pad, D), jnp.float32),
        scratch_types=[
            pltpu.VMEM((chunk,), jnp.int32),
            pltpu.VMEM((chunk,), jnp.int32),
            pltpu.VMEM((chunk,), jnp.float32),
            pltpu.VMEM((chunk, D), jnp.float32),
            pltpu.VMEM_SHARED((n_pad, D), jnp.float32),
            pltpu.SemaphoreType.DMA,
        ],
    )
    def k(table_hbm, srcoff_hbm, dst_hbm, w_hbm, out_hbm,
          idx_v, dst_v, w_v, rows_v, acc_sh, sem):
        c = lax.axis_index("c")
        s = lax.axis_index("s")
        _fill2d(rows_v, chunk, D, 0.0)
        r0 = s * rpt
        for z in range(rpt // chunk):
            pltpu.sync_copy(rows_v, acc_sh.at[pl.ds(r0 + z * chunk, chunk)])
        if rpt % chunk:
            pltpu.sync_copy(rows_v.at[pl.ds(0, rpt % chunk)],
                            acc_sh.at[pl.ds(r0 + rpt - rpt % chunk, rpt % chunk)])
        plsc.subcore_barrier()
        tile_base = ((c * NS + s) if split_edges else s) * ept

        def body(kk, carry):
            base = tile_base + kk * chunk
            pltpu.sync_copy(srcoff_hbm.at[pl.ds(c * e + base, chunk)], idx_v)
            pltpu.async_copy(table_hbm.at[idx_v], rows_v, sem).wait()
            pltpu.sync_copy(dst_hbm.at[pl.ds(base, chunk)], dst_v)
            if weighted:
                pltpu.sync_copy(w_hbm.at[pl.ds(base, chunk)], w_v)

                def wbody(g, carry2):
                    wv = w_v[pl.ds(g * 16, 16)]
                    for j in range(16):
                        r = g * 16 + j
                        wj = jnp.broadcast_to(wv[j], (16,))
                        for cb in range(D // 16):
                            sl = pl.ds(cb * 16, 16)
                            rows_v[r, sl] = rows_v[r, sl] * wj
                    return carry2

                lax.fori_loop(0, chunk // 16, wbody, 0)
            pltpu.sync_copy(rows_v, acc_sh.at[dst_v], add=True)
            return carry

        lax.fori_loop(0, n_iters, body, 0)
        plsc.subcore_barrier()
        pltpu.sync_copy(acc_sh.at[pl.ds(r0, rpt)],
                        out_hbm.at[c, pl.ds(r0, rpt)])

    return k


def _mm_body(x_ref, w_ref, o_ref):
    o_ref[...] = jnp.dot(x_ref[...], w_ref[...],
                         preferred_element_type=jnp.float32)


def _mm(x, w, bn=1000):
    n, kk = x.shape
    m = w.shape[1]
    return pl.pallas_call(
        _mm_body,
        grid=(n // bn,),
        in_specs=[pl.BlockSpec((bn, kk), lambda i: (i, 0)),
                  pl.BlockSpec((kk, m), lambda i: (0, 0))],
        out_specs=pl.BlockSpec((bn, m), lambda i: (i, 0)),
        out_shape=jax.ShapeDtypeStruct((n, m), jnp.float32),
    )(x, w)


def _hpq_body(pre_ref, wab_ref, h_ref, pq_ref):
    pre = pre_ref[...]
    nrm = jnp.sqrt(jnp.sum(pre * pre, axis=-1, keepdims=True))
    h = pre / jnp.maximum(nrm, 1e-12)
    h_ref[...] = h
    pq_ref[...] = jnp.dot(h, wab_ref[...], preferred_element_type=jnp.float32)


def _hpq(pre, wab, bn=1000):
    n, m = pre.shape
    return pl.pallas_call(
        _hpq_body,
        grid=(n // bn,),
        in_specs=[pl.BlockSpec((bn, m), lambda i: (i, 0)),
                  pl.BlockSpec((m, 2), lambda i: (0, 0))],
        out_specs=[pl.BlockSpec((bn, m), lambda i: (i, 0)),
                   pl.BlockSpec((bn, 2), lambda i: (i, 0))],
        out_shape=[jax.ShapeDtypeStruct((n, m), jnp.float32),
                   jax.ShapeDtypeStruct((n, 2), jnp.float32)],
    )(pre, wab)


def _relupq_body(x_ref, w_ref, b_ref, wab_ref, y_ref, pq_ref):
    y = jnp.maximum(jnp.dot(x_ref[...], w_ref[...],
                            preferred_element_type=jnp.float32) + b_ref[...], 0.0)
    y_ref[...] = y
    pq_ref[...] = jnp.dot(y, wab_ref[...], preferred_element_type=jnp.float32)


def _relupq(x, w, b2d, wab, bn=1000):
    n, kk = x.shape
    m = w.shape[1]
    return pl.pallas_call(
        _relupq_body,
        grid=(n // bn,),
        in_specs=[pl.BlockSpec((bn, kk), lambda i: (i, 0)),
                  pl.BlockSpec((kk, m), lambda i: (0, 0)),
                  pl.BlockSpec((1, m), lambda i: (0, 0)),
                  pl.BlockSpec((m, 2), lambda i: (0, 0))],
        out_specs=[pl.BlockSpec((bn, m), lambda i: (i, 0)),
                   pl.BlockSpec((bn, 2), lambda i: (i, 0))],
        out_shape=[jax.ShapeDtypeStruct((n, m), jnp.float32),
                   jax.ShapeDtypeStruct((n, 2), jnp.float32)],
    )(x, w, b2d, wab)


def kernel(emb, gcn1_W, gcn1_b, node1_W, node1_b, edge1_w, edge1_b, edge2_w,
           edge2_b, gcn2_W, gcn2_b, edge_index):
    n = emb.shape[0]
    h_dim = gcn1_W.shape[1]
    src = edge_index[0]
    dst = edge_index[1]
    e = src.shape[0]
    n_pad = ((n + 16 * NS - 1) // (16 * NS)) * (16 * NS)
    if n_pad == n:
        n_pad += 16 * NS

    chunk = 320
    e_pad = ((e + 32 * chunk - 1) // (32 * chunk)) * (32 * chunk)
    npad_e = e_pad - e
    pad_src = jnp.arange(npad_e, dtype=jnp.int32) % n
    pad_dst = n + jnp.arange(npad_e, dtype=jnp.int32) % (n_pad - n)
    src_p = jnp.concatenate([src, pad_src])
    dst_p = jnp.concatenate([dst, pad_dst])
    zpad = jnp.zeros((npad_e,), jnp.float32)

    def padw(w):
        return jnp.concatenate([w, zpad])

    srcoff = jnp.concatenate([src_p, src_p + n])
    srcoff_plain = jnp.concatenate([src_p, src_p])

    degcount = _make_degcount(e, n_pad)
    gate1 = _make_gate(n, e, False, n_pad)
    gate2 = _make_gate(n, e, True, n_pad)
    rp_unw = _make_rowpass(n_pad, e_pad, weighted=False, split_edges=False, chunk=chunk)
    rp_w = _make_rowpass(n_pad, e_pad, weighted=True, split_edges=False, chunk=chunk)
    rp_w_es = _make_rowpass(n_pad, e_pad, weighted=True, split_edges=True, chunk=chunk)
    wdummy = jnp.zeros((e_pad,), jnp.float32)

    def stackhalves(x):
        return jnp.concatenate([x[:, :D], x[:, D:]], axis=0)

    def cathalves(o):
        return jnp.concatenate([o[0, :n], o[1, :n]], axis=1)

    cntp = degcount(dst)
    dinv = lax.rsqrt(cntp[:n] + cntp[n_pad:n_pad + n] + 1.0)
    xw1 = _mm(emb, gcn1_W)
    u = xw1 * dinv[:, None]
    s1 = cathalves(rp_unw(stackhalves(u), srcoff, dst_p, wdummy))
    pre = dinv[:, None] * s1 + (dinv * dinv)[:, None] * xw1 + gcn1_b

    wab1 = jnp.stack([edge1_w[:h_dim], edge1_w[h_dim:]], axis=1)
    h, pq1 = _hpq(pre, wab1)
    b16_1 = jnp.full((16,), edge1_b[0], jnp.float32)
    (ev1,) = gate1(pq1[:, 0], pq1[:, 1], b16_1, src, dst)

    a2 = cathalves(rp_w(stackhalves(h), srcoff, dst_p, padw(ev1)))
    wab2 = jnp.stack([edge2_w[:h_dim], edge2_w[h_dim:]], axis=1)
    na, pq2 = _relupq(a2, node1_W, node1_b[None, :], wab2)

    b16_2 = jnp.full((16,), edge2_b[0], jnp.float32)
    ev2, deg2p = gate2(pq2[:, 0], pq2[:, 1], b16_2, src, dst)
    dinv2 = lax.rsqrt(deg2p[:n] + deg2p[n_pad:n_pad + n] + 1.0)

    ev2p = padw(ev2)
    a3 = cathalves(rp_w(stackhalves(na), srcoff, dst_p, ev2p))
    h2, _ = _relupq(a3, node1_W, node1_b[None, :], wab2)

    xw2 = _mm(h2, gcn2_W)
    u2 = xw2 * dinv2[:, None]
    s4p = rp_w_es(u2, srcoff_plain, dst_p, ev2p)
    s4 = s4p[0, :n] + s4p[1, :n]
    out = dinv2[:, None] * s4 + (dinv2 * dinv2)[:, None] * xw2 + gcn2_b
    return out


# speedup vs baseline: 12.8209x; 1.1284x over previous
import functools

import jax
import jax.numpy as jnp
from jax import lax
from jax.experimental import pallas as pl
from jax.experimental.pallas import tpu as pltpu
from jax.experimental.pallas import tpu_sc as plsc

NC = 2
NS = 16
D = 128


def _fill1d(ref, n, val):
    v = jnp.full((16,), val, jnp.float32)

    def body(i, carry):
        ref[pl.ds(i * 16, 16)] = v
        return carry

    lax.fori_loop(0, n // 16, body, 0)


def _fill2d(ref, rows, cols, val):
    v = jnp.full((16,), val, jnp.float32)

    def body(i, carry):
        for cb in range(cols // 16):
            ref[i, pl.ds(cb * 16, 16)] = v
        return carry

    lax.fori_loop(0, rows, body, 0)


def _make_degcount(e, n_pad):
    epw = e // (NC * NS)
    per_tile = n_pad // NS
    mesh = plsc.VectorSubcoreMesh(core_axis_name="c", subcore_axis_name="s")

    @functools.partial(
        pl.kernel, mesh=mesh,
        out_type=jax.ShapeDtypeStruct((NC * n_pad,), jnp.float32),
        scratch_types=[
            pltpu.VMEM((epw,), jnp.int32),
            pltpu.VMEM((epw,), jnp.float32),
            pltpu.VMEM((per_tile,), jnp.float32),
            pltpu.VMEM_SHARED((n_pad,), jnp.float32),
        ],
    )
    def k(dst_hbm, out_hbm, dst_v, ones_v, zed_v, acc_sh):
        c = lax.axis_index("c")
        s = lax.axis_index("s")
        wid = c * NS + s
        _fill1d(zed_v, per_tile, 0.0)
        pltpu.sync_copy(zed_v, acc_sh.at[pl.ds(s * per_tile, per_tile)])
        _fill1d(ones_v, epw, 1.0)
        plsc.subcore_barrier()
        base = wid * epw
        pltpu.sync_copy(dst_hbm.at[pl.ds(base, epw)], dst_v)
        pltpu.sync_copy(ones_v, acc_sh.at[dst_v], add=True)
        plsc.subcore_barrier()
        pltpu.sync_copy(acc_sh.at[pl.ds(s * per_tile, per_tile)],
                        out_hbm.at[pl.ds(c * n_pad + s * per_tile, per_tile)])

    return k


def _make_gate(n, e, with_deg, n_pad):
    epw = e // (NC * NS)
    per_tile = n_pad // NS
    mesh = plsc.VectorSubcoreMesh(core_axis_name="c", subcore_axis_name="s")
    out_type = [jax.ShapeDtypeStruct((e,), jnp.float32)]
    del n
    scratch = [
        pltpu.VMEM((epw,), jnp.int32),
        pltpu.VMEM((epw,), jnp.int32),
        pltpu.VMEM((epw,), jnp.float32),
        pltpu.VMEM((epw,), jnp.float32),
        pltpu.VMEM((epw,), jnp.float32),
        pltpu.VMEM((16,), jnp.float32),
        pltpu.SemaphoreType.DMA,
    ]
    if with_deg:
        out_type.append(jax.ShapeDtypeStruct((NC * n_pad,), jnp.float32))
        scratch.append(pltpu.VMEM((per_tile,), jnp.float32))
        scratch.append(pltpu.VMEM_SHARED((n_pad,), jnp.float32))

    @functools.partial(pl.kernel, mesh=mesh, out_type=tuple(out_type),
                       scratch_types=scratch)
    def k(p_hbm, q_hbm, b_hbm, src_hbm, dst_hbm, *rest):
        if with_deg:
            (ev_hbm, deg_hbm, src_v, dst_v, pv_v, qv_v, ev_v, b_v, sem,
             zed_v, acc_sh) = rest
        else:
            ev_hbm, src_v, dst_v, pv_v, qv_v, ev_v, b_v, sem = rest
        c = lax.axis_index("c")
        s = lax.axis_index("s")
        wid = c * NS + s
        base = wid * epw
        if with_deg:
            _fill1d(zed_v, per_tile, 0.0)
            pltpu.sync_copy(zed_v, acc_sh.at[pl.ds(s * per_tile, per_tile)])
        pltpu.sync_copy(b_hbm, b_v)
        pltpu.sync_copy(src_hbm.at[pl.ds(base, epw)], src_v)
        pltpu.sync_copy(dst_hbm.at[pl.ds(base, epw)], dst_v)
        pltpu.async_copy(p_hbm.at[src_v], pv_v, sem).wait()
        pltpu.async_copy(q_hbm.at[dst_v], qv_v, sem).wait()
        bb = b_v[...]

        def body(g, carry):
            sl = pl.ds(g * 16, 16)
            x = pv_v[sl] + qv_v[sl] + bb
            ev_v[sl] = 1.0 / (1.0 + jnp.exp(-x))
            return carry

        lax.fori_loop(0, epw // 16, body, 0)
        pltpu.sync_copy(ev_v, ev_hbm.at[pl.ds(base, epw)])
        if with_deg:
            plsc.subcore_barrier()
            pltpu.sync_copy(ev_v, acc_sh.at[dst_v], add=True)
            plsc.subcore_barrier()
            pltpu.sync_copy(acc_sh.at[pl.ds(s * per_tile, per_tile)],
                            deg_hbm.at[pl.ds(c * n_pad + s * per_tile, per_tile)])

    return k


def _make_rowpass(n_pad, e, weighted, split_edges, chunk=160):
    ept = e // (NC * NS) if split_edges else e // NS
    n_iters = ept // chunk
    assert n_iters % 2 == 0
    rpt = n_pad // NS
    mesh = plsc.VectorSubcoreMesh(core_axis_name="c", subcore_axis_name="s")

    @functools.partial(
        pl.kernel, mesh=mesh,
        out_type=jax.ShapeDtypeStruct((NC, n_pad, D), jnp.float32),
        scratch_types=[
            pltpu.VMEM((chunk,), jnp.int32),
            pltpu.VMEM((chunk,), jnp.int32),
            pltpu.VMEM((chunk,), jnp.int32),
            pltpu.VMEM((chunk,), jnp.int32),
            pltpu.VMEM((chunk,), jnp.float32),
            pltpu.VMEM((chunk,), jnp.float32),
            pltpu.VMEM((chunk, D), jnp.float32),
            pltpu.VMEM((chunk, D), jnp.float32),
            pltpu.VMEM_SHARED((n_pad, D), jnp.float32),
            pltpu.SemaphoreType.DMA,
            pltpu.SemaphoreType.DMA,
        ],
    )
    def k(table_hbm, srcoff_hbm, dst_hbm, w_hbm, out_hbm,
          idx0, idx1, dst0, dst1, w0, w1, rows0, rows1, acc_sh, sem0, sem1):
        c = lax.axis_index("c")
        s = lax.axis_index("s")
        idx = (idx0, idx1)
        dstb = (dst0, dst1)
        wb = (w0, w1)
        rows = (rows0, rows1)
        sems = (sem0, sem1)
        _fill2d(rows0, chunk, D, 0.0)
        r0 = s * rpt
        for z in range(rpt // chunk):
            pltpu.sync_copy(rows0, acc_sh.at[pl.ds(r0 + z * chunk, chunk)])
        if rpt % chunk:
            pltpu.sync_copy(rows0.at[pl.ds(0, rpt % chunk)],
                            acc_sh.at[pl.ds(r0 + rpt - rpt % chunk, rpt % chunk)])
        plsc.subcore_barrier()
        tile_base = ((c * NS + s) if split_edges else s) * ept

        pltpu.sync_copy(srcoff_hbm.at[pl.ds(c * e + tile_base, chunk)], idx0)
        pltpu.async_copy(table_hbm.at[idx0], rows0, sem0)

        def step(buf, kk):
            o = 1 - buf
            base = tile_base + kk * chunk

            @pl.when(kk + 1 < n_iters)
            def _():
                pltpu.sync_copy(
                    srcoff_hbm.at[pl.ds(c * e + base + chunk, chunk)], idx[o])
                pltpu.async_copy(table_hbm.at[idx[o]], rows[o], sems[o])

            pltpu.sync_copy(dst_hbm.at[pl.ds(base, chunk)], dstb[buf])
            if weighted:
                pltpu.sync_copy(w_hbm.at[pl.ds(base, chunk)], wb[buf])
            pltpu.make_async_copy(table_hbm.at[idx[buf]], rows[buf],
                                  sems[buf]).wait()
            if weighted:
                def wbody(g, carry2):
                    wv = wb[buf][pl.ds(g * 16, 16)]
                    for j in range(16):
                        r = g * 16 + j
                        wj = jnp.broadcast_to(wv[j], (16,))
                        for cb in range(D // 16):
                            sl = pl.ds(cb * 16, 16)
                            rows[buf][r, sl] = rows[buf][r, sl] * wj
                    return carry2

                lax.fori_loop(0, chunk // 16, wbody, 0)
            pltpu.sync_copy(rows[buf], acc_sh.at[dstb[buf]], add=True)

        def body(k2, carry):
            step(0, 2 * k2)
            step(1, 2 * k2 + 1)
            return carry

        lax.fori_loop(0, n_iters // 2, body, 0)
        plsc.subcore_barrier()
        pltpu.sync_copy(acc_sh.at[pl.ds(r0, rpt)],
                        out_hbm.at[c, pl.ds(r0, rpt)])

    return k


def _mm_body(x_ref, w_ref, o_ref):
    o_ref[...] = jnp.dot(x_ref[...], w_ref[...],
                         preferred_element_type=jnp.float32)


def _mm(x, w, bn=1000):
    n, kk = x.shape
    m = w.shape[1]
    return pl.pallas_call(
        _mm_body,
        grid=(n // bn,),
        in_specs=[pl.BlockSpec((bn, kk), lambda i: (i, 0)),
                  pl.BlockSpec((kk, m), lambda i: (0, 0))],
        out_specs=pl.BlockSpec((bn, m), lambda i: (i, 0)),
        out_shape=jax.ShapeDtypeStruct((n, m), jnp.float32),
    )(x, w)


def _hpq_body(pre_ref, wab_ref, h_ref, pq_ref):
    pre = pre_ref[...]
    nrm = jnp.sqrt(jnp.sum(pre * pre, axis=-1, keepdims=True))
    h = pre / jnp.maximum(nrm, 1e-12)
    h_ref[...] = h
    pq_ref[...] = jnp.dot(h, wab_ref[...], preferred_element_type=jnp.float32)


def _hpq(pre, wab, bn=1000):
    n, m = pre.shape
    return pl.pallas_call(
        _hpq_body,
        grid=(n // bn,),
        in_specs=[pl.BlockSpec((bn, m), lambda i: (i, 0)),
                  pl.BlockSpec((m, 2), lambda i: (0, 0))],
        out_specs=[pl.BlockSpec((bn, m), lambda i: (i, 0)),
                   pl.BlockSpec((bn, 2), lambda i: (i, 0))],
        out_shape=[jax.ShapeDtypeStruct((n, m), jnp.float32),
                   jax.ShapeDtypeStruct((n, 2), jnp.float32)],
    )(pre, wab)


def _relupq_body(x_ref, w_ref, b_ref, wab_ref, y_ref, pq_ref):
    y = jnp.maximum(jnp.dot(x_ref[...], w_ref[...],
                            preferred_element_type=jnp.float32) + b_ref[...], 0.0)
    y_ref[...] = y
    pq_ref[...] = jnp.dot(y, wab_ref[...], preferred_element_type=jnp.float32)


def _relupq(x, w, b2d, wab, bn=1000):
    n, kk = x.shape
    m = w.shape[1]
    return pl.pallas_call(
        _relupq_body,
        grid=(n // bn,),
        in_specs=[pl.BlockSpec((bn, kk), lambda i: (i, 0)),
                  pl.BlockSpec((kk, m), lambda i: (0, 0)),
                  pl.BlockSpec((1, m), lambda i: (0, 0)),
                  pl.BlockSpec((m, 2), lambda i: (0, 0))],
        out_specs=[pl.BlockSpec((bn, m), lambda i: (i, 0)),
                   pl.BlockSpec((bn, 2), lambda i: (i, 0))],
        out_shape=[jax.ShapeDtypeStruct((n, m), jnp.float32),
                   jax.ShapeDtypeStruct((n, 2), jnp.float32)],
    )(x, w, b2d, wab)


def kernel(emb, gcn1_W, gcn1_b, node1_W, node1_b, edge1_w, edge1_b, edge2_w,
           edge2_b, gcn2_W, gcn2_b, edge_index):
    n = emb.shape[0]
    h_dim = gcn1_W.shape[1]
    src = edge_index[0]
    dst = edge_index[1]
    e = src.shape[0]
    n_pad = ((n + 16 * NS - 1) // (16 * NS)) * (16 * NS)
    if n_pad == n:
        n_pad += 16 * NS

    chunk = 160
    e_pad = ((e + 64 * chunk - 1) // (64 * chunk)) * (64 * chunk)
    npad_e = e_pad - e
    pad_src = jnp.arange(npad_e, dtype=jnp.int32) % n
    pad_dst = n + jnp.arange(npad_e, dtype=jnp.int32) % (n_pad - n)
    src_p = jnp.concatenate([src, pad_src])
    dst_p = jnp.concatenate([dst, pad_dst])
    zpad = jnp.zeros((npad_e,), jnp.float32)

    def padw(w):
        return jnp.concatenate([w, zpad])

    srcoff = jnp.concatenate([src_p, src_p + n])
    srcoff_plain = jnp.concatenate([src_p, src_p])

    degcount = _make_degcount(e, n_pad)
    gate1 = _make_gate(n, e, False, n_pad)
    gate2 = _make_gate(n, e, True, n_pad)
    rp_unw = _make_rowpass(n_pad, e_pad, weighted=False, split_edges=False, chunk=chunk)
    rp_w = _make_rowpass(n_pad, e_pad, weighted=True, split_edges=False, chunk=chunk)
    rp_w_es = _make_rowpass(n_pad, e_pad, weighted=True, split_edges=True, chunk=chunk)
    wdummy = jnp.zeros((e_pad,), jnp.float32)

    def stackhalves(x):
        return jnp.concatenate([x[:, :D], x[:, D:]], axis=0)

    def cathalves(o):
        return jnp.concatenate([o[0, :n], o[1, :n]], axis=1)

    cntp = degcount(dst)
    dinv = lax.rsqrt(cntp[:n] + cntp[n_pad:n_pad + n] + 1.0)
    xw1 = _mm(emb, gcn1_W)
    u = xw1 * dinv[:, None]
    s1 = cathalves(rp_unw(stackhalves(u), srcoff, dst_p, wdummy))
    pre = dinv[:, None] * s1 + (dinv * dinv)[:, None] * xw1 + gcn1_b

    wab1 = jnp.stack([edge1_w[:h_dim], edge1_w[h_dim:]], axis=1)
    h, pq1 = _hpq(pre, wab1)
    b16_1 = jnp.full((16,), edge1_b[0], jnp.float32)
    (ev1,) = gate1(pq1[:, 0], pq1[:, 1], b16_1, src, dst)

    a2 = cathalves(rp_w(stackhalves(h), srcoff, dst_p, padw(ev1)))
    wab2 = jnp.stack([edge2_w[:h_dim], edge2_w[h_dim:]], axis=1)
    na, pq2 = _relupq(a2, node1_W, node1_b[None, :], wab2)

    b16_2 = jnp.full((16,), edge2_b[0], jnp.float32)
    ev2, deg2p = gate2(pq2[:, 0], pq2[:, 1], b16_2, src, dst)
    dinv2 = lax.rsqrt(deg2p[:n] + deg2p[n_pad:n_pad + n] + 1.0)

    ev2p = padw(ev2)
    a3 = cathalves(rp_w(stackhalves(na), srcoff, dst_p, ev2p))
    h2, _ = _relupq(a3, node1_W, node1_b[None, :], wab2)

    xw2 = _mm(h2, gcn2_W)
    u2 = xw2 * dinv2[:, None]
    s4p = rp_w_es(u2, srcoff_plain, dst_p, ev2p)
    s4 = s4p[0, :n] + s4p[1, :n]
    out = dinv2[:, None] * s4 + (dinv2 * dinv2)[:, None] * xw2 + gcn2_b
    return out
